# Initial kernel scaffold; baseline (speedup 1.0000x reference)
import jax
import jax.numpy as jnp
from jax.experimental import pallas as pl


def kernel(x):
    raise NotImplementedError("write your pallas kernel here")


# speedup vs baseline: 6.9876x; 6.9876x over previous
import jax
import jax.numpy as jnp
from jax.experimental import pallas as pl
from jax.experimental.pallas import tpu as pltpu

_N, _E, _K = 16384, 64, 8
_ROWS = 1024
_GRID = _N // _ROWS

_KS0 = 0
_KS1 = 42
_KS2 = 42 ^ 0x1BD11BDA
_R1 = (13, 15, 26, 6)
_R2 = (17, 29, 16, 24)


def _rotl(v, d):
    return (v << jnp.uint32(d)) | (v >> jnp.uint32(32 - d))


def _threefry_bits(f):
    ks = (jnp.uint32(_KS0), jnp.uint32(_KS1), jnp.uint32(_KS2))
    x0 = jnp.zeros_like(f) + ks[0]
    x1 = f + ks[1]
    rots = (_R1, _R2, _R1, _R2, _R1)
    inj = ((1, 2), (2, 0), (0, 1), (1, 2), (2, 0))
    for g in range(5):
        for d in rots[g]:
            x0 = x0 + x1
            x1 = _rotl(x1, d)
            x1 = x1 ^ x0
        a, b = inj[g]
        x0 = x0 + ks[a]
        x1 = x1 + ks[b] + jnp.uint32(g + 1)
    return x0 ^ x1


def _body(mask_ref, sm_ref, imp_ref, load_ref):
    g = pl.program_id(0)
    r = jax.lax.broadcasted_iota(jnp.uint32, (_ROWS, _E), 0)
    e = jax.lax.broadcasted_iota(jnp.uint32, (_ROWS, _E), 1)
    base = (g * _ROWS * _E).astype(jnp.uint32)
    f = base + r * jnp.uint32(_E) + e

    bits = _threefry_bits(f)
    mant = (bits >> jnp.uint32(9)).astype(jnp.int32)
    prob = jax.lax.bitcast_convert_type(
        (bits >> jnp.uint32(9)) | jnp.uint32(0x3F800000), jnp.float32
    ) - jnp.float32(1.0)

    keys = mant * 64 + (63 - e.astype(jnp.int32))
    cur = keys
    for _ in range(_K - 1):
        m = jnp.max(cur, axis=1, keepdims=True)
        cur = jnp.where(cur == m, -1, cur)
    t8 = jnp.max(cur, axis=1, keepdims=True)
    maskf = (keys >= t8).astype(jnp.float32)

    ex = jnp.exp(prob)
    den = jnp.sum(ex, axis=1, keepdims=True)
    sm = ex / den

    mask_ref[...] = maskf
    sm_ref[...] = sm

    @pl.when(g == 0)
    def _init():
        imp_ref[...] = jnp.zeros_like(imp_ref)
        load_ref[...] = jnp.zeros_like(load_ref)

    imp_ref[...] += jnp.sum(maskf, axis=0, keepdims=True)
    load_ref[...] += jnp.sum(sm, axis=0, keepdims=True)


def kernel(x):
    del x
    mask, sm, imp, load = pl.pallas_call(
        _body,
        grid=(_GRID,),
        out_specs=(
            pl.BlockSpec((_ROWS, _E), lambda g: (g, 0)),
            pl.BlockSpec((_ROWS, _E), lambda g: (g, 0)),
            pl.BlockSpec((1, _E), lambda g: (0, 0)),
            pl.BlockSpec((1, _E), lambda g: (0, 0)),
        ),
        out_shape=(
            jax.ShapeDtypeStruct((_N, _E), jnp.float32),
            jax.ShapeDtypeStruct((_N, _E), jnp.float32),
            jax.ShapeDtypeStruct((1, _E), jnp.float32),
            jax.ShapeDtypeStruct((1, _E), jnp.float32),
        ),
        compiler_params=pltpu.CompilerParams(
            dimension_semantics=("arbitrary",),
        ),
    )()
    return mask, sm, imp.reshape(_E), load.reshape(_E)
